# Initial kernel scaffold; baseline (speedup 1.0000x reference)
import jax
import jax.numpy as jnp
from jax.experimental import pallas as pl


def kernel(node_specie, node_feats, edge_attrs, edge_feats, senders, receivers, W_sc, W_pre, W1, W2, W3, W4, W_proj, W_mix):
    raise NotImplementedError("write your pallas kernel here")


# speedup vs baseline: 11.3725x; 11.3725x over previous
import functools
import math

import jax
import jax.numpy as jnp
from jax import lax
from jax.experimental import pallas as pl
from jax.experimental.pallas import tpu as pltpu
import jax.experimental.pallas.tpu_sc as plsc

N = 10000
E = 160000
C = 8
D = 8
A = 4
FE = 8
T = 8
M = D * (1 + A)
NUM_SPECIES = 10
AVG_NUM_NEIGHBORS = 16.0

F = C * D
FP = 128

NW = 32
SUB = 128
EPW = 5120
EPAD = NW * EPW
NSUB = EPW // SUB
NBLK = SUB // 16
NPAD = 10112
ROWS_PER_TILE = NPAD // 16


def _silu(x):
  return x * (1.0 / (1.0 + jnp.exp(-x)))


def _node_prep_body(specie_ref, nf_ref, kpre_ref, ksc_ref, x_ref, sc_ref):
  nf = nf_ref[...]
  x_ref[...] = jnp.dot(nf, kpre_ref[...], preferred_element_type=jnp.float32)
  sp = specie_ref[...]
  ksc = ksc_ref[...]
  acc = jnp.zeros_like(nf)
  for s in range(NUM_SPECIES):
    m = (sp == s).astype(jnp.float32)
    acc = acc + m * jnp.dot(nf, ksc[s], preferred_element_type=jnp.float32)
  sc_ref[...] = acc


def _node_prep(specie2d, nf_flat, kpre, ksc):
  Bn = 2000
  grid = (N // Bn,)
  return pl.pallas_call(
      _node_prep_body,
      grid=grid,
      in_specs=[
          pl.BlockSpec((Bn, 1), lambda i: (i, 0)),
          pl.BlockSpec((Bn, F), lambda i: (i, 0)),
          pl.BlockSpec((F, F), lambda i: (0, 0)),
          pl.BlockSpec((NUM_SPECIES, F, F), lambda i: (0, 0, 0)),
      ],
      out_specs=[
          pl.BlockSpec((Bn, F), lambda i: (i, 0)),
          pl.BlockSpec((Bn, F), lambda i: (i, 0)),
      ],
      out_shape=[
          jax.ShapeDtypeStruct((N, F), jnp.float32),
          jax.ShapeDtypeStruct((N, F), jnp.float32),
      ],
  )(specie2d, nf_flat, kpre, ksc)


def _edge_prep_body(ef_ref, ea_ref, w1_ref, w2_ref, w3_ref, w4_ref,
                    s0_ref, s1_ref, q_ref):
  h = _silu(jnp.dot(ef_ref[...], w1_ref[...], preferred_element_type=jnp.float32)
            * (1.0 / math.sqrt(FE)))
  h = _silu(jnp.dot(h, w2_ref[...], preferred_element_type=jnp.float32) * 0.125)
  h = _silu(jnp.dot(h, w3_ref[...], preferred_element_type=jnp.float32) * 0.125)
  mix = jnp.dot(h, w4_ref[...], preferred_element_type=jnp.float32) * 0.125
  ea = ea_ref[...]
  s1 = s1_ref[...]
  q = jnp.dot(mix, s0_ref[...], preferred_element_type=jnp.float32)
  for a in range(A):
    q = q + jnp.dot(ea[:, a:a + 1] * mix, s1[a],
                    preferred_element_type=jnp.float32)
  q_ref[...] = q


def _edge_prep(ef_p, ea_p, w1, w2, w3, w4, s0, s1):
  Be = 2048
  grid = (EPAD // Be,)
  return pl.pallas_call(
      _edge_prep_body,
      grid=grid,
      in_specs=[
          pl.BlockSpec((Be, FE), lambda i: (i, 0)),
          pl.BlockSpec((Be, A), lambda i: (i, 0)),
          pl.BlockSpec((FE, 64), lambda i: (0, 0)),
          pl.BlockSpec((64, 64), lambda i: (0, 0)),
          pl.BlockSpec((64, 64), lambda i: (0, 0)),
          pl.BlockSpec((64, M), lambda i: (0, 0)),
          pl.BlockSpec((M, F), lambda i: (0, 0)),
          pl.BlockSpec((A, M, F), lambda i: (0, 0, 0)),
      ],
      out_specs=pl.BlockSpec((Be, F), lambda i: (i, 0)),
      out_shape=jax.ShapeDtypeStruct((EPAD, F), jnp.float32),
  )(ef_p, ea_p, w1, w2, w3, w4, s0, s1)


def _splat(v):
  return jnp.full((16,), v, jnp.int32)


def _sc_body(x_hbm, q_hbm, snd_hbm, rcv_hbm, z_hbm, out_hbm,
             acc_sp, snd_v, rcv_v, q_v, xs_v, pj_v, gsem):
  sid = lax.axis_index("s")
  core = lax.axis_index("c")
  rows0 = sid * ROWS_PER_TILE

  pltpu.sync_copy(z_hbm.at[pl.ds(rows0, ROWS_PER_TILE)],
                  acc_sp.at[pl.ds(rows0, ROWS_PER_TILE)])
  zero16 = jnp.zeros((16,), jnp.float32)

  def zrow(r, carry):
    for cc in range(4):
      pj_v[r, pl.ds(F + cc * 16, 16)] = zero16
    return carry

  lax.fori_loop(0, SUB, zrow, 0)
  plsc.subcore_barrier()

  base = (core * 16 + sid) * EPW

  def blk_body(b, carry):
    rows = b * 16 + lax.iota(jnp.int32, 16)
    for dh in range(2):
      for th in range(2):
        qv = [[plsc.load_gather(q_v, [rows, _splat((dh * 4 + d) * 8 + th * 4 + t)])
               for t in range(4)] for d in range(4)]
        for c in range(8):
          xs = [plsc.load_gather(xs_v, [rows, _splat(c * 8 + dh * 4 + d)])
                for d in range(4)]
          for t in range(4):
            acc = xs[0] * qv[0][t]
            acc = acc + xs[1] * qv[1][t]
            acc = acc + xs[2] * qv[2][t]
            acc = acc + xs[3] * qv[3][t]
            col = _splat(c * 8 + th * 4 + t)
            if dh == 0:
              plsc.store_scatter(pj_v, [rows, col], acc)
            else:
              plsc.addupdate_scatter(pj_v, [rows, col], acc)
    return carry

  def sub_body(j, carry):
    off = base + j * SUB
    pltpu.sync_copy(snd_hbm.at[pl.ds(off, SUB)], snd_v)
    pltpu.sync_copy(rcv_hbm.at[pl.ds(off, SUB)], rcv_v)
    pltpu.sync_copy(q_hbm.at[pl.ds(off, SUB)], q_v)
    pltpu.async_copy(x_hbm.at[snd_v], xs_v, gsem).wait()
    lax.fori_loop(0, NBLK, blk_body, 0)
    pltpu.sync_copy(pj_v, acc_sp.at[rcv_v], add=True)
    return carry

  lax.fori_loop(0, NSUB, sub_body, 0)

  plsc.subcore_barrier()
  pltpu.sync_copy(acc_sp.at[pl.ds(rows0, ROWS_PER_TILE)],
                  out_hbm.at[core].at[pl.ds(rows0, ROWS_PER_TILE)])


def _sc_aggregate(x_flat, q_flat, snd_p, rcv_p, zeros_nf):
  mesh = plsc.VectorSubcoreMesh(core_axis_name="c", subcore_axis_name="s")
  k = pl.kernel(
      _sc_body,
      out_type=jax.ShapeDtypeStruct((2, NPAD, FP), jnp.float32),
      mesh=mesh,
      compiler_params=pltpu.CompilerParams(needs_layout_passes=False),
      scratch_types=[
          pltpu.VMEM_SHARED((NPAD, FP), jnp.float32),
          pltpu.VMEM((SUB,), jnp.int32),
          pltpu.VMEM((SUB,), jnp.int32),
          pltpu.VMEM((SUB, F), jnp.float32),
          pltpu.VMEM((SUB, FP), jnp.float32),
          pltpu.VMEM((SUB, FP), jnp.float32),
          pltpu.SemaphoreType.DMA,
      ],
  )
  return k(x_flat, q_flat, snd_p, rcv_p, zeros_nf)


def _post_body(agg_ref, kmix_ref, out_ref):
  s = agg_ref[0] + agg_ref[1]
  out_ref[...] = jnp.dot(s, kmix_ref[...], preferred_element_type=jnp.float32)


def _post(agg2, kmix):
  Bn = 2000
  grid = (N // Bn,)
  return pl.pallas_call(
      _post_body,
      grid=grid,
      in_specs=[
          pl.BlockSpec((2, Bn, F), lambda i: (0, i, 0)),
          pl.BlockSpec((F, F), lambda i: (0, 0)),
      ],
      out_specs=pl.BlockSpec((Bn, F), lambda i: (i, 0)),
      out_shape=jax.ShapeDtypeStruct((N, F), jnp.float32),
  )(agg2, kmix)


@jax.jit
def kernel(node_specie, node_feats, edge_attrs, edge_feats, senders, receivers,
           W_sc, W_pre, W1, W2, W3, W4, W_proj, W_mix):
  eye = jnp.eye(D, dtype=jnp.float32)
  inv_sqrt_c = 1.0 / math.sqrt(C)
  kpre = jnp.kron(W_pre, eye) * inv_sqrt_c
  ksc = jax.vmap(lambda w: jnp.kron(w, eye))(W_sc) * inv_sqrt_c
  kmix = jnp.kron(W_mix, eye) * inv_sqrt_c

  wp = W_proj * (1.0 / (math.sqrt(M) * math.sqrt(AVG_NUM_NEIGHBORS)))
  d_idx = jnp.arange(D)
  t_idx = jnp.arange(T)
  cols = d_idx[:, None] * T + t_idx[None, :]
  s0 = jnp.zeros((M, F), jnp.float32).at[d_idx[:, None], cols].set(wp[:D])
  s1_list = []
  for a in range(A):
    rows = D + d_idx * A + a
    s1_list.append(
        jnp.zeros((M, F), jnp.float32).at[rows[:, None], cols].set(wp[rows]))
  s1 = jnp.stack(s1_list)

  nf_flat = node_feats.reshape(N, F)
  specie2d = node_specie.reshape(N, 1).astype(jnp.int32)
  x_flat, sc_flat = _node_prep(specie2d, nf_flat, kpre, ksc)

  pad = EPAD - E
  ef_p = jnp.pad(edge_feats, ((0, pad), (0, 0)))
  ea_p = jnp.pad(edge_attrs, ((0, pad), (0, 0)))
  q_flat = _edge_prep(ef_p, ea_p, W1, W2, W3, W4, s0, s1)

  snd_p = jnp.pad(senders.astype(jnp.int32), (0, pad))
  rcv_p = jnp.pad(receivers.astype(jnp.int32), (0, pad))
  zeros_nf = jnp.zeros((NPAD, FP), jnp.float32)
  x_pad = jnp.pad(x_flat, ((0, NPAD - N), (0, FP - F)))
  agg2 = _sc_aggregate(x_pad, q_flat, snd_p, rcv_p, zeros_nf)

  out_flat = _post(agg2[:, :N, :F], kmix)
  return (out_flat.reshape(N, C, T), sc_flat.reshape(N, C, D))


# speedup vs baseline: 13.5885x; 1.1949x over previous
import functools
import math

import jax
import jax.numpy as jnp
from jax import lax
from jax.experimental import pallas as pl
from jax.experimental.pallas import tpu as pltpu
import jax.experimental.pallas.tpu_sc as plsc

N = 10000
E = 160000
C = 8
D = 8
A = 4
FE = 8
T = 8
M = D * (1 + A)
NUM_SPECIES = 10
AVG_NUM_NEIGHBORS = 16.0

F = C * D
FP = 128

NW = 32
SUB = 64
EPW = 5120
EPAD = NW * EPW
NSUB = EPW // SUB
NBLK = SUB // 16
EPAD2 = EPAD + 2048
NPAD = 10112
ROWS_PER_TILE = NPAD // 16


def _silu(x):
  return x * (1.0 / (1.0 + jnp.exp(-x)))


def _node_prep_body(specie_ref, nf_ref, kpre_ref, ksc_ref, x_ref, sc_ref):
  nf = nf_ref[...]
  x_ref[...] = jnp.dot(nf, kpre_ref[...], preferred_element_type=jnp.float32)
  sp = specie_ref[...]
  ksc = ksc_ref[...]
  acc = jnp.zeros_like(nf)
  for s in range(NUM_SPECIES):
    m = (sp == s).astype(jnp.float32)
    acc = acc + m * jnp.dot(nf, ksc[s], preferred_element_type=jnp.float32)
  sc_ref[...] = acc


def _node_prep(specie2d, nf_flat, kpre, ksc):
  Bn = 2000
  grid = (N // Bn,)
  return pl.pallas_call(
      _node_prep_body,
      grid=grid,
      in_specs=[
          pl.BlockSpec((Bn, 1), lambda i: (i, 0)),
          pl.BlockSpec((Bn, F), lambda i: (i, 0)),
          pl.BlockSpec((F, F), lambda i: (0, 0)),
          pl.BlockSpec((NUM_SPECIES, F, F), lambda i: (0, 0, 0)),
      ],
      out_specs=[
          pl.BlockSpec((Bn, F), lambda i: (i, 0)),
          pl.BlockSpec((Bn, F), lambda i: (i, 0)),
      ],
      out_shape=[
          jax.ShapeDtypeStruct((N, F), jnp.float32),
          jax.ShapeDtypeStruct((N, F), jnp.float32),
      ],
  )(specie2d, nf_flat, kpre, ksc)


def _edge_prep_body(ef_ref, ea_ref, w1_ref, w2_ref, w3_ref, w4_ref,
                    s0_ref, s1_ref, q_ref):
  h = _silu(jnp.dot(ef_ref[...], w1_ref[...], preferred_element_type=jnp.float32)
            * (1.0 / math.sqrt(FE)))
  h = _silu(jnp.dot(h, w2_ref[...], preferred_element_type=jnp.float32) * 0.125)
  h = _silu(jnp.dot(h, w3_ref[...], preferred_element_type=jnp.float32) * 0.125)
  mix = jnp.dot(h, w4_ref[...], preferred_element_type=jnp.float32) * 0.125
  ea = ea_ref[...]
  s1 = s1_ref[...]
  q = jnp.dot(mix, s0_ref[...], preferred_element_type=jnp.float32)
  for a in range(A):
    q = q + jnp.dot(ea[:, a:a + 1] * mix, s1[a],
                    preferred_element_type=jnp.float32)
  q_ref[...] = q


def _edge_prep(ef_p, ea_p, w1, w2, w3, w4, s0, s1):
  Be = 2048
  grid = (EPAD2 // Be,)
  return pl.pallas_call(
      _edge_prep_body,
      grid=grid,
      in_specs=[
          pl.BlockSpec((Be, FE), lambda i: (i, 0)),
          pl.BlockSpec((Be, A), lambda i: (i, 0)),
          pl.BlockSpec((FE, 64), lambda i: (0, 0)),
          pl.BlockSpec((64, 64), lambda i: (0, 0)),
          pl.BlockSpec((64, 64), lambda i: (0, 0)),
          pl.BlockSpec((64, M), lambda i: (0, 0)),
          pl.BlockSpec((M, F), lambda i: (0, 0)),
          pl.BlockSpec((A, M, F), lambda i: (0, 0, 0)),
      ],
      out_specs=pl.BlockSpec((Be, F), lambda i: (i, 0)),
      out_shape=jax.ShapeDtypeStruct((EPAD2, F), jnp.float32),
  )(ef_p, ea_p, w1, w2, w3, w4, s0, s1)


def _splat(v):
  return jnp.full((16,), v, jnp.int32)


def _sc_body(x_hbm, q_hbm, snd_hbm, rcv_hbm, z_hbm, out_hbm,
             acc_sp, snd0, snd1, rcv0, rcv1, q0, q1, xs0, xs1, pj,
             sem_snd0, sem_snd1, sem_q0, sem_q1, sem_rcv0, sem_rcv1,
             gsem0, gsem1, ssem):
  sid = lax.axis_index("s")
  core = lax.axis_index("c")
  rows0 = sid * ROWS_PER_TILE
  base = (core * 16 + sid) * EPW

  snd_v = [snd0, snd1]
  rcv_v = [rcv0, rcv1]
  q_v = [q0, q1]
  xs_v = [xs0, xs1]
  sem_snd = [sem_snd0, sem_snd1]
  sem_q = [sem_q0, sem_q1]
  sem_rcv = [sem_rcv0, sem_rcv1]
  gsem = [gsem0, gsem1]

  def lin_issue(j, p):
    off = base + j * SUB
    pltpu.async_copy(snd_hbm.at[pl.ds(off, SUB)], snd_v[p], sem_snd[p])
    pltpu.async_copy(q_hbm.at[pl.ds(off, SUB)], q_v[p], sem_q[p])

  def lin_wait(j, p):
    off = base + j * SUB
    pltpu.make_async_copy(snd_hbm.at[pl.ds(off, SUB)], snd_v[p], sem_snd[p]).wait()
    pltpu.make_async_copy(q_hbm.at[pl.ds(off, SUB)], q_v[p], sem_q[p]).wait()

  def rcv_issue(j, p):
    off = base + j * SUB
    pltpu.async_copy(rcv_hbm.at[pl.ds(off, SUB)], rcv_v[p], sem_rcv[p])

  def rcv_wait(j, p):
    off = base + j * SUB
    pltpu.make_async_copy(rcv_hbm.at[pl.ds(off, SUB)], rcv_v[p], sem_rcv[p]).wait()

  def gather_issue(p):
    pltpu.async_copy(x_hbm.at[snd_v[p]], xs_v[p], gsem[p])

  def gather_wait(p):
    pltpu.make_async_copy(x_hbm.at[snd_v[p]], xs_v[p], gsem[p]).wait()

  def scat_issue(p):
    pltpu.async_copy(pj, acc_sp.at[rcv_v[p]], ssem, add=True)

  def scat_wait(p):
    pltpu.make_async_copy(pj, acc_sp.at[rcv_v[p]], ssem).wait()

  def compute(p):
    qr = q_v[p]
    xr = xs_v[p]
    pr = pj

    def blk_body(b, carry):
      rows = b * 16 + lax.iota(jnp.int32, 16)
      for dh in range(2):
        for th in range(2):
          qv = [[plsc.load_gather(qr, [rows, _splat((dh * 4 + d) * 8 + th * 4 + t)])
                 for t in range(4)] for d in range(4)]
          for c in range(8):
            xs = [plsc.load_gather(xr, [rows, _splat(c * 8 + dh * 4 + d)])
                  for d in range(4)]
            for t in range(4):
              acc = xs[0] * qv[0][t]
              acc = acc + xs[1] * qv[1][t]
              acc = acc + xs[2] * qv[2][t]
              acc = acc + xs[3] * qv[3][t]
              col = _splat(c * 8 + th * 4 + t)
              if dh == 0:
                plsc.store_scatter(pr, [rows, col], acc)
              else:
                plsc.addupdate_scatter(pr, [rows, col], acc)
      return carry

    lax.fori_loop(0, NBLK, blk_body, 0)

  pltpu.sync_copy(z_hbm.at[pl.ds(rows0, ROWS_PER_TILE)],
                  acc_sp.at[pl.ds(rows0, ROWS_PER_TILE)])
  zero16 = jnp.zeros((16,), jnp.float32)

  def zrow(r, carry):
    for cc in range(4):
      pj[r, pl.ds(F + cc * 16, 16)] = zero16
    return carry

  lax.fori_loop(0, SUB, zrow, 0)

  lin_issue(0, 0)
  rcv_issue(0, 0)
  lin_wait(0, 0)
  gather_issue(0)
  lin_issue(1, 1)
  plsc.subcore_barrier()

  def pair_body(i, carry):
    for p in range(2):
      j = 2 * i + p
      pp = 1 - p
      lin_wait(j + 1, pp)
      gather_issue(pp)
      gather_wait(p)
      if p == 1:
        scat_wait(pp)
      else:
        @pl.when(j > 0)
        def _():
          scat_wait(pp)
      rcv_issue(j + 1, pp)
      compute(p)
      rcv_wait(j, p)
      scat_issue(p)
      lin_issue(j + 2, p)
    return carry

  lax.fori_loop(0, NSUB // 2, pair_body, 0)

  scat_wait(1)
  gather_wait(0)
  lin_wait(NSUB + 1, 1)
  rcv_wait(NSUB, 0)

  plsc.subcore_barrier()
  pltpu.sync_copy(acc_sp.at[pl.ds(rows0, ROWS_PER_TILE)],
                  out_hbm.at[core].at[pl.ds(rows0, ROWS_PER_TILE)])


def _sc_aggregate(x_flat, q_flat, snd_p, rcv_p, zeros_nf):
  mesh = plsc.VectorSubcoreMesh(core_axis_name="c", subcore_axis_name="s")
  k = pl.kernel(
      _sc_body,
      out_type=jax.ShapeDtypeStruct((2, NPAD, FP), jnp.float32),
      mesh=mesh,
      compiler_params=pltpu.CompilerParams(needs_layout_passes=False),
      scratch_types=(
          [pltpu.VMEM_SHARED((NPAD, FP), jnp.float32)]
          + [pltpu.VMEM((SUB,), jnp.int32)] * 4
          + [pltpu.VMEM((SUB, F), jnp.float32)] * 2
          + [pltpu.VMEM((SUB, FP), jnp.float32)] * 3
          + [pltpu.SemaphoreType.DMA] * 9
      ),
  )
  return k(x_flat, q_flat, snd_p, rcv_p, zeros_nf)


def _post_body(agg_ref, kmix_ref, out_ref):
  s = agg_ref[0] + agg_ref[1]
  out_ref[...] = jnp.dot(s, kmix_ref[...], preferred_element_type=jnp.float32)


def _post(agg2, kmix):
  Bn = 2000
  grid = (N // Bn,)
  return pl.pallas_call(
      _post_body,
      grid=grid,
      in_specs=[
          pl.BlockSpec((2, Bn, F), lambda i: (0, i, 0)),
          pl.BlockSpec((F, F), lambda i: (0, 0)),
      ],
      out_specs=pl.BlockSpec((Bn, F), lambda i: (i, 0)),
      out_shape=jax.ShapeDtypeStruct((N, F), jnp.float32),
  )(agg2, kmix)


@jax.jit
def kernel(node_specie, node_feats, edge_attrs, edge_feats, senders, receivers,
           W_sc, W_pre, W1, W2, W3, W4, W_proj, W_mix):
  eye = jnp.eye(D, dtype=jnp.float32)
  inv_sqrt_c = 1.0 / math.sqrt(C)
  kpre = jnp.kron(W_pre, eye) * inv_sqrt_c
  ksc = jax.vmap(lambda w: jnp.kron(w, eye))(W_sc) * inv_sqrt_c
  kmix = jnp.kron(W_mix, eye) * inv_sqrt_c

  wp = W_proj * (1.0 / (math.sqrt(M) * math.sqrt(AVG_NUM_NEIGHBORS)))
  d_idx = jnp.arange(D)
  t_idx = jnp.arange(T)
  cols = d_idx[:, None] * T + t_idx[None, :]
  s0 = jnp.zeros((M, F), jnp.float32).at[d_idx[:, None], cols].set(wp[:D])
  s1_list = []
  for a in range(A):
    rows = D + d_idx * A + a
    s1_list.append(
        jnp.zeros((M, F), jnp.float32).at[rows[:, None], cols].set(wp[rows]))
  s1 = jnp.stack(s1_list)

  nf_flat = node_feats.reshape(N, F)
  specie2d = node_specie.reshape(N, 1).astype(jnp.int32)
  x_flat, sc_flat = _node_prep(specie2d, nf_flat, kpre, ksc)

  pad = EPAD2 - E
  ef_p = jnp.pad(edge_feats, ((0, pad), (0, 0)))
  ea_p = jnp.pad(edge_attrs, ((0, pad), (0, 0)))
  q_flat = _edge_prep(ef_p, ea_p, W1, W2, W3, W4, s0, s1)

  snd_p = jnp.pad(senders.astype(jnp.int32), (0, pad))
  rcv_p = jnp.pad(receivers.astype(jnp.int32), (0, pad))
  zeros_nf = jnp.zeros((NPAD, FP), jnp.float32)
  x_pad = jnp.pad(x_flat, ((0, NPAD - N), (0, FP - F)))
  agg2 = _sc_aggregate(x_pad, q_flat, snd_p, rcv_p, zeros_nf)

  out_flat = _post(agg2[:, :N, :F], kmix)
  return (out_flat.reshape(N, C, T), sc_flat.reshape(N, C, D))


# speedup vs baseline: 14.1081x; 1.0382x over previous
import functools
import math

import jax
import jax.numpy as jnp
from jax import lax
from jax.experimental import pallas as pl
from jax.experimental.pallas import tpu as pltpu
import jax.experimental.pallas.tpu_sc as plsc

N = 10000
E = 160000
C = 8
D = 8
A = 4
FE = 8
T = 8
M = D * (1 + A)
NUM_SPECIES = 10
AVG_NUM_NEIGHBORS = 16.0

F = C * D
FP = 128

NW = 32
SUB = 64
EPW = 5120
EPAD = NW * EPW
NSUB = EPW // SUB
NBLK = SUB // 16
EPAD2 = EPAD + 2048
NPAD = 10112
ROWS_PER_TILE = NPAD // 16


def _silu(x):
  return x * (1.0 / (1.0 + jnp.exp(-x)))


def _node_prep_body(specie_ref, nf_ref, kpre_ref, ksc_ref, x_ref, sc_ref):
  nf = nf_ref[...]
  x_ref[...] = jnp.dot(nf, kpre_ref[...], preferred_element_type=jnp.float32)
  sp = specie_ref[...]
  ksc = ksc_ref[...]
  acc = jnp.zeros_like(nf)
  for s in range(NUM_SPECIES):
    m = (sp == s).astype(jnp.float32)
    acc = acc + m * jnp.dot(nf, ksc[s], preferred_element_type=jnp.float32)
  sc_ref[...] = acc


def _node_prep(specie2d, nf_flat, kpre, ksc):
  Bn = 2000
  grid = (N // Bn,)
  return pl.pallas_call(
      _node_prep_body,
      grid=grid,
      in_specs=[
          pl.BlockSpec((Bn, 1), lambda i: (i, 0)),
          pl.BlockSpec((Bn, F), lambda i: (i, 0)),
          pl.BlockSpec((F, F), lambda i: (0, 0)),
          pl.BlockSpec((NUM_SPECIES, F, F), lambda i: (0, 0, 0)),
      ],
      out_specs=[
          pl.BlockSpec((Bn, F), lambda i: (i, 0)),
          pl.BlockSpec((Bn, F), lambda i: (i, 0)),
      ],
      out_shape=[
          jax.ShapeDtypeStruct((N, F), jnp.float32),
          jax.ShapeDtypeStruct((N, F), jnp.float32),
      ],
  )(specie2d, nf_flat, kpre, ksc)


def _edge_prep_body(ef_ref, ea_ref, w1_ref, w2_ref, w3_ref, w4_ref,
                    s0_ref, s1_ref, q_ref):
  h = _silu(jnp.dot(ef_ref[...], w1_ref[...], preferred_element_type=jnp.float32)
            * (1.0 / math.sqrt(FE)))
  h = _silu(jnp.dot(h, w2_ref[...], preferred_element_type=jnp.float32) * 0.125)
  h = _silu(jnp.dot(h, w3_ref[...], preferred_element_type=jnp.float32) * 0.125)
  mix = jnp.dot(h, w4_ref[...], preferred_element_type=jnp.float32) * 0.125
  ea = ea_ref[...]
  s1 = s1_ref[...]
  q = jnp.dot(mix, s0_ref[...], preferred_element_type=jnp.float32)
  for a in range(A):
    q = q + jnp.dot(ea[:, a:a + 1] * mix, s1[a],
                    preferred_element_type=jnp.float32)
  q_ref[...] = q


def _edge_prep(ef_p, ea_p, w1, w2, w3, w4, s0, s1):
  Be = 2048
  grid = (EPAD // Be,)
  return pl.pallas_call(
      _edge_prep_body,
      grid=grid,
      in_specs=[
          pl.BlockSpec((Be, FE), lambda i: (i, 0)),
          pl.BlockSpec((Be, A), lambda i: (i, 0)),
          pl.BlockSpec((FE, 64), lambda i: (0, 0)),
          pl.BlockSpec((64, 64), lambda i: (0, 0)),
          pl.BlockSpec((64, 64), lambda i: (0, 0)),
          pl.BlockSpec((64, M), lambda i: (0, 0)),
          pl.BlockSpec((M, F), lambda i: (0, 0)),
          pl.BlockSpec((A, M, F), lambda i: (0, 0, 0)),
      ],
      out_specs=pl.BlockSpec((Be, F), lambda i: (i, 0)),
      out_shape=jax.ShapeDtypeStruct((EPAD, F), jnp.float32),
  )(ef_p, ea_p, w1, w2, w3, w4, s0, s1)


def _splat(v):
  return jnp.full((16,), v, jnp.int32)


def _sc_body(x_hbm, q_hbm, snd_hbm, rcv_hbm, z_hbm, out_hbm,
             acc_sp, snd0, snd1, rcv0, rcv1, q0, q1, xs0, xs1, pj,
             sem_snd0, sem_snd1, sem_q0, sem_q1, sem_rcv0, sem_rcv1,
             gsem0, gsem1, ssem):
  sid = lax.axis_index("s")
  core = lax.axis_index("c")
  rows0 = sid * ROWS_PER_TILE
  base = (core * 16 + sid) * EPW

  snd_v = [snd0, snd1]
  rcv_v = [rcv0, rcv1]
  q_v = [q0, q1]
  xs_v = [xs0, xs1]
  sem_snd = [sem_snd0, sem_snd1]
  sem_q = [sem_q0, sem_q1]
  sem_rcv = [sem_rcv0, sem_rcv1]
  gsem = [gsem0, gsem1]

  baseb = (base // SUB) * 8

  def lin_issue(j, p):
    off = base + j * SUB
    boff = baseb + j * 8
    pltpu.async_copy(snd_hbm.at[pl.ds(off, SUB)], snd_v[p], sem_snd[p])
    pltpu.async_copy(q_hbm.at[pl.ds(boff, 8)], q_v[p], sem_q[p])

  def lin_wait(j, p):
    off = base + j * SUB
    boff = baseb + j * 8
    pltpu.make_async_copy(snd_hbm.at[pl.ds(off, SUB)], snd_v[p], sem_snd[p]).wait()
    pltpu.make_async_copy(q_hbm.at[pl.ds(boff, 8)], q_v[p], sem_q[p]).wait()

  def rcv_issue(j, p):
    off = base + j * SUB
    pltpu.async_copy(rcv_hbm.at[pl.ds(off, SUB)], rcv_v[p], sem_rcv[p])

  def rcv_wait(j, p):
    off = base + j * SUB
    pltpu.make_async_copy(rcv_hbm.at[pl.ds(off, SUB)], rcv_v[p], sem_rcv[p]).wait()

  def gather_issue(p):
    pltpu.async_copy(x_hbm.at[snd_v[p]], xs_v[p], gsem[p])

  def gather_wait(p):
    pltpu.make_async_copy(x_hbm.at[snd_v[p]], xs_v[p], gsem[p]).wait()

  def scat_issue(p):
    pltpu.async_copy(pj, acc_sp.at[rcv_v[p]], ssem, add=True)

  def scat_wait(p):
    pltpu.make_async_copy(pj, acc_sp.at[rcv_v[p]], ssem).wait()

  def compute(p):
    qr = q_v[p]
    xr = xs_v[p]
    pr = pj

    def blk_body(b, carry):
      rows = b * 16 + lax.iota(jnp.int32, 16)
      for dh in range(2):
        for th in range(2):
          qv = [[qr[dh * 4 + d, pl.ds((th * 4 + t) * 64 + b * 16, 16)]
                 for t in range(4)] for d in range(4)]
          for c in range(8):
            xs = [plsc.load_gather(xr, [rows, _splat(c * 8 + dh * 4 + d)])
                  for d in range(4)]
            for t in range(4):
              acc = xs[0] * qv[0][t]
              acc = acc + xs[1] * qv[1][t]
              acc = acc + xs[2] * qv[2][t]
              acc = acc + xs[3] * qv[3][t]
              col = _splat(c * 8 + th * 4 + t)
              if dh == 0:
                plsc.store_scatter(pr, [rows, col], acc)
              else:
                plsc.addupdate_scatter(pr, [rows, col], acc)
      return carry

    lax.fori_loop(0, NBLK, blk_body, 0)

  pltpu.sync_copy(z_hbm.at[pl.ds(rows0, ROWS_PER_TILE)],
                  acc_sp.at[pl.ds(rows0, ROWS_PER_TILE)])
  zero16 = jnp.zeros((16,), jnp.float32)

  def zrow(r, carry):
    for cc in range(4):
      pj[r, pl.ds(F + cc * 16, 16)] = zero16
    return carry

  lax.fori_loop(0, SUB, zrow, 0)

  lin_issue(0, 0)
  rcv_issue(0, 0)
  lin_wait(0, 0)
  gather_issue(0)
  lin_issue(1, 1)
  plsc.subcore_barrier()

  nsub = jnp.minimum(NSUB, (E - base + SUB - 1) // SUB)

  def pair_body(i, carry):
    for p in range(2):
      j = 2 * i + p
      pp = 1 - p
      lin_wait(j + 1, pp)
      gather_issue(pp)
      gather_wait(p)
      if p == 1:
        scat_wait(pp)
      else:
        @pl.when(j > 0)
        def _():
          scat_wait(pp)
      rcv_issue(j + 1, pp)
      compute(p)
      rcv_wait(j, p)
      scat_issue(p)
      lin_issue(j + 2, p)
    return carry

  lax.fori_loop(0, nsub // 2, pair_body, 0)

  scat_wait(1)
  gather_wait(0)
  lin_wait(nsub + 1, 1)
  rcv_wait(nsub, 0)

  plsc.subcore_barrier()
  pltpu.sync_copy(acc_sp.at[pl.ds(rows0, ROWS_PER_TILE)],
                  out_hbm.at[core].at[pl.ds(rows0, ROWS_PER_TILE)])


def _sc_aggregate(x_flat, q_flat, snd_p, rcv_p, zeros_nf):
  mesh = plsc.VectorSubcoreMesh(core_axis_name="c", subcore_axis_name="s")
  k = pl.kernel(
      _sc_body,
      out_type=jax.ShapeDtypeStruct((2, NPAD, FP), jnp.float32),
      mesh=mesh,
      compiler_params=pltpu.CompilerParams(needs_layout_passes=False),
      scratch_types=(
          [pltpu.VMEM_SHARED((NPAD, FP), jnp.float32)]
          + [pltpu.VMEM((SUB,), jnp.int32)] * 4
          + [pltpu.VMEM((8, 512), jnp.float32)] * 2
          + [pltpu.VMEM((SUB, FP), jnp.float32)] * 3
          + [pltpu.SemaphoreType.DMA] * 9
      ),
  )
  return k(x_flat, q_flat, snd_p, rcv_p, zeros_nf)


def _post_body(agg_ref, kmix_ref, out_ref):
  s = agg_ref[0] + agg_ref[1]
  out_ref[...] = jnp.dot(s, kmix_ref[...], preferred_element_type=jnp.float32)


def _post(agg2, kmix):
  Bn = 2000
  grid = (N // Bn,)
  return pl.pallas_call(
      _post_body,
      grid=grid,
      in_specs=[
          pl.BlockSpec((2, Bn, F), lambda i: (0, i, 0)),
          pl.BlockSpec((F, F), lambda i: (0, 0)),
      ],
      out_specs=pl.BlockSpec((Bn, F), lambda i: (i, 0)),
      out_shape=jax.ShapeDtypeStruct((N, F), jnp.float32),
  )(agg2, kmix)


@jax.jit
def kernel(node_specie, node_feats, edge_attrs, edge_feats, senders, receivers,
           W_sc, W_pre, W1, W2, W3, W4, W_proj, W_mix):
  eye = jnp.eye(D, dtype=jnp.float32)
  inv_sqrt_c = 1.0 / math.sqrt(C)
  kpre = jnp.kron(W_pre, eye) * inv_sqrt_c
  ksc = jax.vmap(lambda w: jnp.kron(w, eye))(W_sc) * inv_sqrt_c
  kmix = jnp.kron(W_mix, eye) * inv_sqrt_c

  wp = W_proj * (1.0 / (math.sqrt(M) * math.sqrt(AVG_NUM_NEIGHBORS)))
  d_idx = jnp.arange(D)
  t_idx = jnp.arange(T)
  cols = d_idx[:, None] * T + t_idx[None, :]
  s0 = jnp.zeros((M, F), jnp.float32).at[d_idx[:, None], cols].set(wp[:D])
  s1_list = []
  for a in range(A):
    rows = D + d_idx * A + a
    s1_list.append(
        jnp.zeros((M, F), jnp.float32).at[rows[:, None], cols].set(wp[rows]))
  s1 = jnp.stack(s1_list)

  nf_flat = node_feats.reshape(N, F)
  specie2d = node_specie.reshape(N, 1).astype(jnp.int32)
  x_flat, sc_flat = _node_prep(specie2d, nf_flat, kpre, ksc)

  ef_p = jnp.pad(edge_feats, ((0, EPAD - E), (0, 0)))
  ea_p = jnp.pad(edge_attrs, ((0, EPAD - E), (0, 0)))
  q_flat = _edge_prep(ef_p, ea_p, W1, W2, W3, W4, s0, s1)
  q_blk = (q_flat.reshape(EPAD // 64, 64, F).swapaxes(1, 2)
           .reshape(EPAD // 64 * 8, 8 * 64))

  snd_p = jnp.pad(senders.astype(jnp.int32), (0, EPAD2 - E))
  rcv_p = jnp.pad(receivers.astype(jnp.int32), (0, EPAD2 - E))
  zeros_nf = jnp.zeros((NPAD, FP), jnp.float32)
  x_pad = jnp.pad(x_flat, ((0, NPAD - N), (0, FP - F)))
  agg2 = _sc_aggregate(x_pad, q_blk, snd_p, rcv_p, zeros_nf)

  out_flat = _post(agg2[:, :N, :F], kmix)
  return (out_flat.reshape(N, C, T), sc_flat.reshape(N, C, D))
